# Initial kernel scaffold; baseline (speedup 1.0000x reference)
import jax
import jax.numpy as jnp
from jax.experimental import pallas as pl


def kernel(x, edge_index, batch, edge_weight, W_rel1, b_rel1, W_root1, W_rel2, b_rel2, W_root2, W_rel3, b_rel3, W_root3, W_lin, b_lin):
    raise NotImplementedError("write your pallas kernel here")


# speedup vs baseline: 1.0140x; 1.0140x over previous
import functools

import jax
import jax.numpy as jnp
from jax import lax
from jax.experimental import pallas as pl
from jax.experimental.pallas import tpu as pltpu
from jax.experimental.pallas import tpu_sc as plsc

N = 10000
E = 160000
H = 256
C = 10
G = 64

NC = 2
NS = 16
NW = NC * NS
OWN = 320
NPAD = NW * OWN
ACC_ROWS = OWN + 8
EBS = 1600
NBLK = E // EBS
CH = 80
CAP = EBS + CH

_mesh = plsc.VectorSubcoreMesh(core_axis_name="c", subcore_axis_name="s")


def _splat(vec, i):
    return lax.gather(
        vec, jnp.full((16, 1), i, jnp.int32),
        lax.GatherDimensionNumbers(
            offset_dims=(), collapsed_slice_dims=(0,), start_index_map=(0,)),
        slice_sizes=(1,),
        mode=lax.GatherScatterMode.PROMISE_IN_BOUNDS)


@functools.partial(
    pl.kernel,
    out_type=jax.ShapeDtypeStruct((NPAD, H), jnp.float32),
    mesh=_mesh,
    compiler_params=pltpu.CompilerParams(needs_layout_passes=False),
    scratch_types=[
        pltpu.VMEM((EBS,), jnp.int32),
        pltpu.VMEM((EBS,), jnp.int32),
        pltpu.VMEM((EBS,), jnp.float32),
        pltpu.VMEM((CAP,), jnp.int32),
        pltpu.VMEM((CAP,), jnp.int32),
        pltpu.VMEM((CAP,), jnp.float32),
        pltpu.VMEM((CH, H), jnp.float32),
        pltpu.VMEM((ACC_ROWS, H), jnp.float32),
        pltpu.SemaphoreType.DMA,
    ],
)
def _seg256(p_hbm, src_hbm, dst_hbm, ew_hbm, out_hbm,
            src_v, dst_v, ew_v, csrc_v, cdl_v, cw_v, rows_v, acc_v, sem):
    c = lax.axis_index("c")
    s = lax.axis_index("s")
    w = s * NC + c
    lo = w * OWN

    def zrow(i, _):
        for f in range(H // 16):
            acc_v[i, pl.ds(f * 16, 16)] = jnp.zeros((16,), jnp.float32)
        return 0
    lax.fori_loop(0, ACC_ROWS, zrow, 0)

    def block(b, _):
        bb = b * EBS
        pltpu.sync_copy(src_hbm.at[pl.ds(bb, EBS)], src_v)
        pltpu.sync_copy(dst_hbm.at[pl.ds(bb, EBS)], dst_v)
        pltpu.sync_copy(ew_hbm.at[pl.ds(bb, EBS)], ew_v)

        def scan(g, off):
            o = g * 16
            srcv = src_v[pl.ds(o, 16)]
            dstv = dst_v[pl.ds(o, 16)]
            ewv = ew_v[pl.ds(o, 16)]
            inm = (dstv >= lo) & (dstv < lo + OWN)
            plsc.store_compressed(csrc_v.at[pl.ds(off, 16)], srcv, mask=inm)
            plsc.store_compressed(cdl_v.at[pl.ds(off, 16)], dstv - lo, mask=inm)
            plsc.store_compressed(cw_v.at[pl.ds(off, 16)], ewv, mask=inm)
            cnt = jnp.max(plsc.all_reduce_population_count(inm))
            return off + cnt
        off = lax.fori_loop(0, EBS // 16, scan, jnp.int32(0))

        for g in range(CH // 16):
            csrc_v[pl.ds(off + g * 16, 16)] = jnp.full((16,), lo, jnp.int32)
            cdl_v[pl.ds(off + g * 16, 16)] = jnp.full((16,), OWN, jnp.int32)
            cw_v[pl.ds(off + g * 16, 16)] = jnp.zeros((16,), jnp.float32)

        nch = (off + CH - 1) // CH
        def chunk(ch, _):
            cb = ch * CH
            pltpu.async_copy(p_hbm.at[csrc_v.at[pl.ds(cb, CH)]],
                             rows_v, sem).wait()
            for g in range(CH // 16):
                wv = cw_v[pl.ds(cb + g * 16, 16)]
                dlv = cdl_v[pl.ds(cb + g * 16, 16)]
                def edge(e, _):
                    spl = _splat(wv, e)
                    dl = _splat(dlv, e)[0]
                    r = g * 16 + e
                    for f in range(H // 16):
                        acc_v[dl, pl.ds(f * 16, 16)] = (
                            acc_v[dl, pl.ds(f * 16, 16)]
                            + rows_v[r, pl.ds(f * 16, 16)] * spl)
                    return 0
                lax.fori_loop(0, 16, edge, 0)
            return 0
        lax.fori_loop(0, nch, chunk, 0)
        return 0

    lax.fori_loop(0, NBLK, block, 0)

    for j in range(OWN // CH):
        pltpu.sync_copy(acc_v.at[pl.ds(j * CH, CH)],
                        out_hbm.at[pl.ds(lo + j * CH, CH)])


@functools.partial(
    pl.kernel,
    out_type=jax.ShapeDtypeStruct((NPAD,), jnp.float32),
    mesh=_mesh,
    compiler_params=pltpu.CompilerParams(needs_layout_passes=False),
    scratch_types=[
        pltpu.VMEM((EBS,), jnp.int32),
        pltpu.VMEM((EBS,), jnp.int32),
        pltpu.VMEM((EBS,), jnp.float32),
        pltpu.VMEM((CAP,), jnp.int32),
        pltpu.VMEM((CAP,), jnp.int32),
        pltpu.VMEM((CAP,), jnp.float32),
        pltpu.VMEM((CH,), jnp.float32),
        pltpu.VMEM((16, 336), jnp.float32),
        pltpu.VMEM((OWN,), jnp.float32),
        pltpu.SemaphoreType.DMA,
    ],
)
def _seg1(x_hbm, src_hbm, dst_hbm, ew_hbm, out_hbm,
          src_v, dst_v, ew_v, csrc_v, cdl_v, cw_v, xg_v, acc_v, red_v, sem):
    c = lax.axis_index("c")
    s = lax.axis_index("s")
    w = s * NC + c
    lo = w * OWN
    lane = lax.broadcasted_iota(jnp.int32, (16,), 0)

    def zrow(i, _):
        for j in range(336 // 16):
            acc_v[i, pl.ds(j * 16, 16)] = jnp.zeros((16,), jnp.float32)
        return 0
    lax.fori_loop(0, 16, zrow, 0)

    def block(b, _):
        bb = b * EBS
        pltpu.sync_copy(src_hbm.at[pl.ds(bb, EBS)], src_v)
        pltpu.sync_copy(dst_hbm.at[pl.ds(bb, EBS)], dst_v)
        pltpu.sync_copy(ew_hbm.at[pl.ds(bb, EBS)], ew_v)

        def scan(g, off):
            o = g * 16
            srcv = src_v[pl.ds(o, 16)]
            dstv = dst_v[pl.ds(o, 16)]
            ewv = ew_v[pl.ds(o, 16)]
            inm = (dstv >= lo) & (dstv < lo + OWN)
            plsc.store_compressed(csrc_v.at[pl.ds(off, 16)], srcv, mask=inm)
            plsc.store_compressed(cdl_v.at[pl.ds(off, 16)], dstv - lo, mask=inm)
            plsc.store_compressed(cw_v.at[pl.ds(off, 16)], ewv, mask=inm)
            cnt = jnp.max(plsc.all_reduce_population_count(inm))
            return off + cnt
        off = lax.fori_loop(0, EBS // 16, scan, jnp.int32(0))

        for g in range(CH // 16):
            csrc_v[pl.ds(off + g * 16, 16)] = jnp.full((16,), lo, jnp.int32)
            cdl_v[pl.ds(off + g * 16, 16)] = jnp.full((16,), OWN, jnp.int32)
            cw_v[pl.ds(off + g * 16, 16)] = jnp.zeros((16,), jnp.float32)

        nch = (off + CH - 1) // CH
        def chunk(ch, _):
            cb = ch * CH
            pltpu.async_copy(x_hbm.at[csrc_v.at[pl.ds(cb, CH)]],
                             xg_v, sem).wait()
            for g in range(CH // 16):
                xv = xg_v[pl.ds(g * 16, 16)]
                wv = cw_v[pl.ds(cb + g * 16, 16)]
                dlv = cdl_v[pl.ds(cb + g * 16, 16)]
                plsc.addupdate_scatter(acc_v, [lane, dlv], xv * wv)
            return 0
        lax.fori_loop(0, nch, chunk, 0)
        return 0

    lax.fori_loop(0, NBLK, block, 0)

    def red(j, _):
        v = acc_v[0, pl.ds(j * 16, 16)]
        for l in range(1, 16):
            v = v + acc_v[l, pl.ds(j * 16, 16)]
        red_v[pl.ds(j * 16, 16)] = v
        return 0
    lax.fori_loop(0, OWN // 16, red, 0)
    pltpu.sync_copy(red_v, out_hbm.at[pl.ds(lo, OWN)])


_BN = 1000


def _l1_body(a_ref, x_ref, wr_ref, wo_ref, b_ref, o_ref):
    o_ref[...] = jnp.maximum(
        a_ref[...] * wr_ref[...] + x_ref[...] * wo_ref[...] + b_ref[...], 0.0)


_l1_call = pl.pallas_call(
    _l1_body,
    grid=(N // _BN,),
    in_specs=[
        pl.BlockSpec((_BN, 1), lambda i: (i, 0)),
        pl.BlockSpec((_BN, 1), lambda i: (i, 0)),
        pl.BlockSpec((1, H), lambda i: (0, 0)),
        pl.BlockSpec((1, H), lambda i: (0, 0)),
        pl.BlockSpec((1, H), lambda i: (0, 0)),
    ],
    out_specs=pl.BlockSpec((_BN, H), lambda i: (i, 0)),
    out_shape=jax.ShapeDtypeStruct((N, H), jnp.float32),
)


def _lin_body(relu, agg_ref, h_ref, wr_ref, wo_ref, b_ref, o_ref):
    acc = jnp.dot(agg_ref[...], wr_ref[...], preferred_element_type=jnp.float32)
    acc += jnp.dot(h_ref[...], wo_ref[...], preferred_element_type=jnp.float32)
    acc += b_ref[...]
    o_ref[...] = jnp.maximum(acc, 0.0) if relu else acc


def _make_lin(relu):
    return pl.pallas_call(
        functools.partial(_lin_body, relu),
        grid=(N // _BN,),
        in_specs=[
            pl.BlockSpec((_BN, H), lambda i: (i, 0)),
            pl.BlockSpec((_BN, H), lambda i: (i, 0)),
            pl.BlockSpec((H, H), lambda i: (0, 0)),
            pl.BlockSpec((H, H), lambda i: (0, 0)),
            pl.BlockSpec((1, H), lambda i: (0, 0)),
        ],
        out_specs=pl.BlockSpec((_BN, H), lambda i: (i, 0)),
        out_shape=jax.ShapeDtypeStruct((N, H), jnp.float32),
    )


_lin_relu = _make_lin(True)
_lin_nor = _make_lin(False)


def _pool_body(h_ref, bat_ref, wl_ref, bl_ref, o_ref):
    gi = lax.broadcasted_iota(jnp.int32, (1, G), 1)
    oneh = (bat_ref[...] == gi).astype(jnp.float32)
    sums = lax.dot_general(oneh, h_ref[...], (((0,), (0,)), ((), ())),
                           preferred_element_type=jnp.float32)
    cnts = lax.dot_general(oneh, jnp.ones((N, 1), jnp.float32),
                           (((0,), (0,)), ((), ())),
                           preferred_element_type=jnp.float32)
    pooled = sums / jnp.maximum(cnts, 1.0)
    o_ref[...] = jnp.dot(pooled, wl_ref[...],
                         preferred_element_type=jnp.float32) + bl_ref[...]


_pool_call = pl.pallas_call(
    _pool_body,
    out_shape=jax.ShapeDtypeStruct((G, C), jnp.float32),
)


def kernel(x, edge_index, batch, edge_weight,
           W_rel1, b_rel1, W_root1,
           W_rel2, b_rel2, W_root2,
           W_rel3, b_rel3, W_root3,
           W_lin, b_lin):
    src = edge_index[0]
    dst = edge_index[1]

    agg1 = _seg1(x[:, 0], src, dst, edge_weight)[:N]
    h1 = _l1_call(agg1[:, None], x, W_rel1, W_root1, b_rel1[None, :])
    agg2 = _seg256(h1, src, dst, edge_weight)[:N]
    h2 = _lin_relu(agg2, h1, W_rel2, W_root2, b_rel2[None, :])
    agg3 = _seg256(h2, src, dst, edge_weight)[:N]
    h3 = _lin_nor(agg3, h2, W_rel3, W_root3, b_rel3[None, :])
    return _pool_call(h3, batch[:, None], W_lin, b_lin)
